# baseline (speedup 1.0000x reference)
import functools

import jax
import jax.numpy as jnp
from jax import lax
from jax.experimental import pallas as pl
from jax.experimental.pallas import tpu as pltpu
from jax.experimental.pallas import tpu_sc as plsc

VOCAB = 1000000
EMBED_DIM = 64
NUM_CLASS = 4
L = 16384

NC = 2
NS = 16
NW = NC * NS
N_CHUNKS = 4
CHUNK = L // NW // N_CHUNKS

_mesh = plsc.VectorSubcoreMesh(
    core_axis_name="c", subcore_axis_name="s", num_cores=NC, num_subcores=NS
)


@functools.partial(
    pl.kernel,
    out_type=jax.ShapeDtypeStruct((NW, EMBED_DIM), jnp.float32),
    mesh=_mesh,
    scratch_types=[
        pltpu.VMEM((N_CHUNKS, CHUNK), jnp.int32),
        pltpu.VMEM((N_CHUNKS, CHUNK, EMBED_DIM), jnp.float32),
        pltpu.VMEM((EMBED_DIM,), jnp.float32),
        pltpu.SemaphoreType.DMA,
    ],
    compiler_params=pltpu.CompilerParams(use_tc_tiling_on_sc=False),
)
def _sc_partial_sums(idx_hbm, table_hbm, out_hbm, idx_v, rows_v, part_v, sem):
    wid = lax.axis_index("s") * NC + lax.axis_index("c")
    pltpu.sync_copy(idx_hbm.at[wid], idx_v)
    copies = [
        pltpu.async_copy(table_hbm.at[idx_v.at[j]], rows_v.at[j], sem)
        for j in range(N_CHUNKS)
    ]
    for c in copies:
        c.wait()

    def body(r, accs):
        new = []
        for c in range(N_CHUNKS):
            a = accs[c]
            for j in range(N_CHUNKS):
                a = a + rows_v[j, r, pl.ds(c * 16, 16)]
            new.append(a)
        return tuple(new)

    zeros = jnp.zeros((16,), jnp.float32)
    accs = lax.fori_loop(0, CHUNK, body, (zeros,) * N_CHUNKS)
    for c in range(N_CHUNKS):
        part_v[pl.ds(c * 16, 16)] = accs[c]
    pltpu.sync_copy(part_v, out_hbm.at[wid])


def _tc_head(parts_ref, w_ref, b_ref, o_ref):
    s = jnp.sum(parts_ref[...], axis=0, keepdims=True) * (1.0 / L)
    o_ref[...] = jnp.dot(s, w_ref[...].T, preferred_element_type=jnp.float32) + b_ref[...]


def kernel(text, emb_table, fc_w, fc_b):
    idx = text.astype(jnp.int32).reshape(NW, N_CHUNKS, CHUNK)
    parts = _sc_partial_sums(idx, emb_table)
    out = pl.pallas_call(
        _tc_head,
        out_shape=jax.ShapeDtypeStruct((1, NUM_CLASS), jnp.float32),
    )(parts, fc_w, fc_b.reshape(1, NUM_CLASS))
    return out


# speedup vs baseline: 4.7071x; 4.7071x over previous
import functools

import jax
import jax.numpy as jnp
from jax import lax
from jax.experimental import pallas as pl
from jax.experimental.pallas import tpu as pltpu
from jax.experimental.pallas import tpu_sc as plsc

VOCAB = 1000000
EMBED_DIM = 64
NUM_CLASS = 4
L = 16384

NC = 2
NS = 16
NW = NC * NS
N_CHUNKS = 4
CHUNK = L // NW // N_CHUNKS
VOCAB_PAD = 1000064
VSLICE = VOCAB_PAD // NS
ZCHUNK = 500

_mesh = plsc.VectorSubcoreMesh(
    core_axis_name="c", subcore_axis_name="s", num_cores=NC, num_subcores=NS
)


@functools.partial(
    pl.kernel,
    out_type=[
        jax.ShapeDtypeStruct((VOCAB_PAD,), jnp.float32),
        jax.ShapeDtypeStruct((VOCAB_PAD,), jnp.float32),
    ],
    mesh=_mesh,
    scratch_types=[
        pltpu.VMEM((N_CHUNKS, CHUNK), jnp.int32),
        pltpu.VMEM((CHUNK,), jnp.float32),
        pltpu.VMEM((ZCHUNK * 16,), jnp.float32),
        pltpu.VMEM_SHARED((VOCAB_PAD,), jnp.float32),
        pltpu.SemaphoreType.DMA,
    ],
)
def _sc_counts(idx_hbm, out0_hbm, out1_hbm, idx_v, ones_v, zbuf_v, counts_sh, sem):
    cid = lax.axis_index("c")
    sid = lax.axis_index("s")
    wid = sid * NC + cid

    pltpu.sync_copy(idx_hbm.at[wid], idx_v)
    for k in range(CHUNK // 16):
        ones_v[pl.ds(k * 16, 16)] = jnp.ones((16,), jnp.float32)

    def zbody(k, _):
        zbuf_v[pl.ds(k * 16, 16)] = jnp.zeros((16,), jnp.float32)
        return 0

    lax.fori_loop(0, ZCHUNK, zbody, 0)
    base = sid * VSLICE
    for k in range(VSLICE // (ZCHUNK * 16)):
        pltpu.sync_copy(
            zbuf_v, counts_sh.at[pl.ds(base + k * ZCHUNK * 16, ZCHUNK * 16)]
        )
    rem = VSLICE % (ZCHUNK * 16)
    if rem:
        pltpu.sync_copy(
            zbuf_v.at[pl.ds(0, rem)],
            counts_sh.at[pl.ds(base + VSLICE - rem, rem)],
        )
    plsc.subcore_barrier()

    for j in range(N_CHUNKS):
        pltpu.sync_copy(ones_v, counts_sh.at[idx_v.at[j]], add=True)
    plsc.subcore_barrier()

    def export_chunk(off, n):
        pltpu.sync_copy(counts_sh.at[pl.ds(base + off, n)], zbuf_v.at[pl.ds(0, n)])

        @pl.when(cid == 0)
        def _():
            pltpu.sync_copy(zbuf_v.at[pl.ds(0, n)], out0_hbm.at[pl.ds(base + off, n)])

        @pl.when(cid == 1)
        def _():
            pltpu.sync_copy(zbuf_v.at[pl.ds(0, n)], out1_hbm.at[pl.ds(base + off, n)])

    for k in range(VSLICE // (ZCHUNK * 16)):
        export_chunk(k * ZCHUNK * 16, ZCHUNK * 16)
    rem = VSLICE % (ZCHUNK * 16)
    if rem:
        export_chunk(VSLICE - rem, rem)


SWEEP_BLK = 16384
SWEEP_STEPS = pl.cdiv(VOCAB, SWEEP_BLK)


def _tc_sweep(tt_ref, c0_ref, c1_ref, w_ref, b_ref, o_ref, acc_ref):
    pid = pl.program_id(0)

    @pl.when(pid == 0)
    def _():
        acc_ref[...] = jnp.zeros_like(acc_ref)

    cb = c0_ref[...] + c1_ref[...]
    prod = tt_ref[...] * cb[None, :]
    cols = pid * SWEEP_BLK + lax.broadcasted_iota(jnp.int32, (1, SWEEP_BLK), 1)
    prod = jnp.where(cols < VOCAB, prod, 0.0)
    acc_ref[...] += jnp.sum(prod, axis=1).reshape(1, EMBED_DIM)

    @pl.when(pid == SWEEP_STEPS - 1)
    def _():
        emb = acc_ref[...] * (1.0 / L)
        o_ref[...] = (
            jnp.dot(emb, w_ref[...].T, preferred_element_type=jnp.float32)
            + b_ref[...]
        )


def kernel(text, emb_table, fc_w, fc_b):
    idx = text.astype(jnp.int32).reshape(NW, N_CHUNKS, CHUNK)
    c0, c1 = _sc_counts(idx)
    tt = emb_table.T
    out = pl.pallas_call(
        _tc_sweep,
        grid=(SWEEP_STEPS,),
        in_specs=[
            pl.BlockSpec((EMBED_DIM, SWEEP_BLK), lambda i: (0, i)),
            pl.BlockSpec((SWEEP_BLK,), lambda i: (i,)),
            pl.BlockSpec((SWEEP_BLK,), lambda i: (i,)),
            pl.BlockSpec((NUM_CLASS, EMBED_DIM), lambda i: (0, 0)),
            pl.BlockSpec((1, NUM_CLASS), lambda i: (0, 0)),
        ],
        out_specs=pl.BlockSpec((1, NUM_CLASS), lambda i: (0, 0)),
        out_shape=jax.ShapeDtypeStruct((1, NUM_CLASS), jnp.float32),
        scratch_shapes=[pltpu.VMEM((1, EMBED_DIM), jnp.float32)],
    )(tt, c0, c1, fc_w, fc_b.reshape(1, NUM_CLASS))
    return out


# speedup vs baseline: 5.4016x; 1.1475x over previous
import functools

import jax
import jax.numpy as jnp
from jax import lax
from jax.experimental import pallas as pl
from jax.experimental.pallas import tpu as pltpu
from jax.experimental.pallas import tpu_sc as plsc

VOCAB = 1000000
EMBED_DIM = 64
NUM_CLASS = 4
L = 16384

NC = 2
NS = 16
NW = NC * NS
N_CHUNKS = 4
CHUNK = L // NW // N_CHUNKS
VOCAB_PAD = 1000064
VSLICE = VOCAB_PAD // NS
ZCHUNK = 500

_mesh = plsc.VectorSubcoreMesh(
    core_axis_name="c", subcore_axis_name="s", num_cores=NC, num_subcores=NS
)


@functools.partial(
    pl.kernel,
    out_type=[
        jax.ShapeDtypeStruct((VOCAB_PAD,), jnp.float32),
        jax.ShapeDtypeStruct((VOCAB_PAD,), jnp.float32),
    ],
    mesh=_mesh,
    scratch_types=[
        pltpu.VMEM((N_CHUNKS, CHUNK), jnp.int32),
        pltpu.VMEM((CHUNK,), jnp.float32),
        pltpu.VMEM((ZCHUNK * 16,), jnp.float32),
        pltpu.VMEM_SHARED((VOCAB_PAD,), jnp.float32),
        pltpu.SemaphoreType.DMA,
    ],
)
def _sc_counts(idx_hbm, out0_hbm, out1_hbm, idx_v, ones_v, zbuf_v, counts_sh, sem):
    cid = lax.axis_index("c")
    sid = lax.axis_index("s")
    wid = sid * NC + cid

    pltpu.sync_copy(idx_hbm.at[wid], idx_v)
    for k in range(CHUNK // 16):
        ones_v[pl.ds(k * 16, 16)] = jnp.ones((16,), jnp.float32)

    def zbody(k, _):
        zbuf_v[pl.ds(k * 16, 16)] = jnp.zeros((16,), jnp.float32)
        return 0

    lax.fori_loop(0, ZCHUNK, zbody, 0)
    base = sid * VSLICE
    for k in range(VSLICE // (ZCHUNK * 16)):
        pltpu.sync_copy(
            zbuf_v, counts_sh.at[pl.ds(base + k * ZCHUNK * 16, ZCHUNK * 16)]
        )
    rem = VSLICE % (ZCHUNK * 16)
    if rem:
        pltpu.sync_copy(
            zbuf_v.at[pl.ds(0, rem)],
            counts_sh.at[pl.ds(base + VSLICE - rem, rem)],
        )
    plsc.subcore_barrier()

    for j in range(N_CHUNKS):
        pltpu.sync_copy(ones_v, counts_sh.at[idx_v.at[j]], add=True)
    plsc.subcore_barrier()

    def export_chunk(off, n):
        pltpu.sync_copy(counts_sh.at[pl.ds(base + off, n)], zbuf_v.at[pl.ds(0, n)])

        @pl.when(cid == 0)
        def _():
            pltpu.sync_copy(zbuf_v.at[pl.ds(0, n)], out0_hbm.at[pl.ds(base + off, n)])

        @pl.when(cid == 1)
        def _():
            pltpu.sync_copy(zbuf_v.at[pl.ds(0, n)], out1_hbm.at[pl.ds(base + off, n)])

    for k in range(VSLICE // (ZCHUNK * 16)):
        export_chunk(k * ZCHUNK * 16, ZCHUNK * 16)
    rem = VSLICE % (ZCHUNK * 16)
    if rem:
        export_chunk(VSLICE - rem, rem)


SWEEP_BLK = 32768
SWEEP_STEPS = pl.cdiv(VOCAB, SWEEP_BLK)


def _tc_sweep(tt_ref, c0_ref, c1_ref, w_ref, b_ref, o_ref, acc_ref):
    pid = pl.program_id(0)

    @pl.when(pid == 0)
    def _():
        acc_ref[...] = jnp.zeros_like(acc_ref)

    cb = c0_ref[...] + c1_ref[...]
    prod = tt_ref[...] * cb[None, :]
    cols = pid * SWEEP_BLK + lax.broadcasted_iota(jnp.int32, (1, SWEEP_BLK), 1)
    prod = jnp.where(cols < VOCAB, prod, 0.0)
    acc_ref[...] += jnp.sum(prod, axis=1).reshape(1, EMBED_DIM)

    @pl.when(pid == SWEEP_STEPS - 1)
    def _():
        emb = acc_ref[...] * (1.0 / L)
        o_ref[...] = (
            jnp.dot(emb, w_ref[...].T, preferred_element_type=jnp.float32)
            + b_ref[...]
        )


def kernel(text, emb_table, fc_w, fc_b):
    idx = text.astype(jnp.int32).reshape(NW, N_CHUNKS, CHUNK)
    c0, c1 = _sc_counts(idx)
    tt = emb_table.T
    out = pl.pallas_call(
        _tc_sweep,
        grid=(SWEEP_STEPS,),
        in_specs=[
            pl.BlockSpec((EMBED_DIM, SWEEP_BLK), lambda i: (0, i)),
            pl.BlockSpec((SWEEP_BLK,), lambda i: (i,)),
            pl.BlockSpec((SWEEP_BLK,), lambda i: (i,)),
            pl.BlockSpec((NUM_CLASS, EMBED_DIM), lambda i: (0, 0)),
            pl.BlockSpec((1, NUM_CLASS), lambda i: (0, 0)),
        ],
        out_specs=pl.BlockSpec((1, NUM_CLASS), lambda i: (0, 0)),
        out_shape=jax.ShapeDtypeStruct((1, NUM_CLASS), jnp.float32),
        scratch_shapes=[pltpu.VMEM((1, EMBED_DIM), jnp.float32)],
    )(tt, c0, c1, fc_w, fc_b.reshape(1, NUM_CLASS))
    return out


# speedup vs baseline: 5.4537x; 1.0097x over previous
import functools

import jax
import jax.numpy as jnp
from jax import lax
from jax.experimental import pallas as pl
from jax.experimental.pallas import tpu as pltpu
from jax.experimental.pallas import tpu_sc as plsc

VOCAB = 1000000
EMBED_DIM = 64
NUM_CLASS = 4
L = 16384

NC = 2
NS = 16
NW = NC * NS
N_CHUNKS = 4
CHUNK = L // NW // N_CHUNKS
VOCAB_PAD = 1000064
VSLICE = VOCAB_PAD // NS
ZCHUNK = 500

_mesh = plsc.VectorSubcoreMesh(
    core_axis_name="c", subcore_axis_name="s", num_cores=NC, num_subcores=NS
)


@functools.partial(
    pl.kernel,
    out_type=[
        jax.ShapeDtypeStruct((VOCAB_PAD,), jnp.float32),
        jax.ShapeDtypeStruct((VOCAB_PAD,), jnp.float32),
    ],
    mesh=_mesh,
    scratch_types=[
        pltpu.VMEM((N_CHUNKS, CHUNK), jnp.int32),
        pltpu.VMEM((CHUNK,), jnp.float32),
        pltpu.VMEM((ZCHUNK * 16,), jnp.float32),
        pltpu.VMEM_SHARED((VOCAB_PAD,), jnp.float32),
        pltpu.SemaphoreType.DMA,
    ],
)
def _sc_counts(idx_hbm, out0_hbm, out1_hbm, idx_v, ones_v, zbuf_v, counts_sh, sem):
    cid = lax.axis_index("c")
    sid = lax.axis_index("s")
    wid = sid * NC + cid

    pltpu.sync_copy(idx_hbm.at[wid], idx_v)
    for k in range(CHUNK // 16):
        ones_v[pl.ds(k * 16, 16)] = jnp.ones((16,), jnp.float32)

    def zbody(k, _):
        zbuf_v[pl.ds(k * 16, 16)] = jnp.zeros((16,), jnp.float32)
        return 0

    lax.fori_loop(0, ZCHUNK, zbody, 0)
    base = sid * VSLICE
    for k in range(VSLICE // (ZCHUNK * 16)):
        pltpu.sync_copy(
            zbuf_v, counts_sh.at[pl.ds(base + k * ZCHUNK * 16, ZCHUNK * 16)]
        )
    rem = VSLICE % (ZCHUNK * 16)
    if rem:
        pltpu.sync_copy(
            zbuf_v.at[pl.ds(0, rem)],
            counts_sh.at[pl.ds(base + VSLICE - rem, rem)],
        )
    plsc.subcore_barrier()

    for j in range(N_CHUNKS):
        pltpu.sync_copy(ones_v, counts_sh.at[idx_v.at[j]], add=True)
    plsc.subcore_barrier()

    def export_chunk(off, n):
        pltpu.sync_copy(counts_sh.at[pl.ds(base + off, n)], zbuf_v.at[pl.ds(0, n)])

        @pl.when(cid == 0)
        def _():
            pltpu.sync_copy(zbuf_v.at[pl.ds(0, n)], out0_hbm.at[pl.ds(base + off, n)])

        @pl.when(cid == 1)
        def _():
            pltpu.sync_copy(zbuf_v.at[pl.ds(0, n)], out1_hbm.at[pl.ds(base + off, n)])

    for k in range(VSLICE // (ZCHUNK * 16)):
        export_chunk(k * ZCHUNK * 16, ZCHUNK * 16)
    rem = VSLICE % (ZCHUNK * 16)
    if rem:
        export_chunk(VSLICE - rem, rem)


SWEEP_BLK = 65536
SWEEP_STEPS = pl.cdiv(VOCAB, SWEEP_BLK)


def _tc_sweep(tt_ref, c0_ref, c1_ref, w_ref, b_ref, o_ref, acc_ref):
    pid = pl.program_id(0)

    @pl.when(pid == 0)
    def _():
        acc_ref[...] = jnp.zeros_like(acc_ref)

    cb = c0_ref[...] + c1_ref[...]
    prod = tt_ref[...] * cb[None, :]
    cols = pid * SWEEP_BLK + lax.broadcasted_iota(jnp.int32, (1, SWEEP_BLK), 1)
    prod = jnp.where(cols < VOCAB, prod, 0.0)
    acc_ref[...] += jnp.sum(prod, axis=1).reshape(1, EMBED_DIM)

    @pl.when(pid == SWEEP_STEPS - 1)
    def _():
        emb = acc_ref[...] * (1.0 / L)
        o_ref[...] = (
            jnp.dot(emb, w_ref[...].T, preferred_element_type=jnp.float32)
            + b_ref[...]
        )


def kernel(text, emb_table, fc_w, fc_b):
    idx = text.astype(jnp.int32).reshape(NW, N_CHUNKS, CHUNK)
    c0, c1 = _sc_counts(idx)
    tt = emb_table.T
    out = pl.pallas_call(
        _tc_sweep,
        grid=(SWEEP_STEPS,),
        in_specs=[
            pl.BlockSpec((EMBED_DIM, SWEEP_BLK), lambda i: (0, i)),
            pl.BlockSpec((SWEEP_BLK,), lambda i: (i,)),
            pl.BlockSpec((SWEEP_BLK,), lambda i: (i,)),
            pl.BlockSpec((NUM_CLASS, EMBED_DIM), lambda i: (0, 0)),
            pl.BlockSpec((1, NUM_CLASS), lambda i: (0, 0)),
        ],
        out_specs=pl.BlockSpec((1, NUM_CLASS), lambda i: (0, 0)),
        out_shape=jax.ShapeDtypeStruct((1, NUM_CLASS), jnp.float32),
        scratch_shapes=[pltpu.VMEM((1, EMBED_DIM), jnp.float32)],
    )(tt, c0, c1, fc_w, fc_b.reshape(1, NUM_CLASS))
    return out
